# Initial kernel scaffold; baseline (speedup 1.0000x reference)
import jax
import jax.numpy as jnp
from jax.experimental import pallas as pl


def kernel(img, entity_table, color_table):
    raise NotImplementedError("write your pallas kernel here")


# speedup vs baseline: 4.0403x; 4.0403x over previous
import functools

import jax
import jax.numpy as jnp
from jax import lax
from jax.experimental import pallas as pl
from jax.experimental.pallas import tpu as pltpu
from jax.experimental.pallas import tpu_sc as plsc

_NUM_TILES = 16
_NUM_COLORS = 12
_EMB = 16

_N = 16384 * 9 * 9
_NW = 32
_BW = _N // _NW
_C = 1152
_NCHUNK = _BW // _C
_NSUB = _C // 128
_NGRP = _C // 16


def _body(img_ref, tcomb_ref, out_ref, pairs_ref, comb_ref, rows_ref, sem):
    wid = lax.axis_index("s") * 2 + lax.axis_index("c")
    iota = lax.iota(jnp.int32, 16)

    def chunk(k, _):
        base = wid * _BW + k * _C
        pltpu.sync_copy(img_ref.at[pl.ds(2 * base, 2 * _C)], pairs_ref)
        for i in range(_NGRP):
            e = plsc.load_gather(pairs_ref, [32 * i + 2 * iota])
            c = plsc.load_gather(pairs_ref, [32 * i + 2 * iota + 1])
            comb_ref[i // 8, pl.ds(16 * (i % 8), 16)] = e * _NUM_COLORS + c
        copies = []
        for j in range(_NSUB):
            copies.append(pltpu.make_async_copy(
                tcomb_ref.at[comb_ref.at[j]],
                rows_ref.at[pl.ds(128 * j, 128)], sem))
        for cp in copies:
            cp.start()
        for cp in copies:
            cp.wait()
        pltpu.sync_copy(rows_ref, out_ref.at[pl.ds(base, _C)])
        return ()

    lax.fori_loop(0, _NCHUNK, chunk, ())


@jax.jit
def _encode(img_flat, tcomb):
    mesh = plsc.VectorSubcoreMesh(core_axis_name="c", subcore_axis_name="s")
    return pl.kernel(
        _body,
        out_type=jax.ShapeDtypeStruct((_N, 2 * _EMB), jnp.float32),
        mesh=mesh,
        scratch_types=[
            pltpu.VMEM((2 * _C,), jnp.int32),
            pltpu.VMEM((_NSUB, 128), jnp.int32),
            pltpu.VMEM((_C, 2 * _EMB), jnp.float32),
            pltpu.SemaphoreType.DMA,
        ],
        compiler_params=pltpu.CompilerParams(
            needs_layout_passes=False, use_tc_tiling_on_sc=False),
    )(img_flat, tcomb)


def kernel(img, entity_table, color_table):
    tcomb = jnp.concatenate(
        [jnp.repeat(entity_table, _NUM_COLORS, axis=0),
         jnp.tile(color_table, (_NUM_TILES, 1))], axis=1)
    out = _encode(img.reshape(-1), tcomb)
    return out.reshape(img.shape[:-1] + (2 * _EMB,))


# speedup vs baseline: 4.5842x; 1.1346x over previous
import jax
import jax.numpy as jnp
from jax import lax
from jax.experimental import pallas as pl
from jax.experimental.pallas import tpu as pltpu
from jax.experimental.pallas import tpu_sc as plsc

_NUM_TILES = 16
_NUM_COLORS = 12
_COMB = _NUM_TILES * _NUM_COLORS
_EMB = 16

_N = 16384 * 9 * 9
_NW = 32
_BW = _N // _NW
_C = 1152
_NCHUNK = _BW // _C
_NSUB = _C // 128
_NGRP = _C // 16
_NBUF = 2


def _body(img_ref, tcomb_ref, out_ref, pairs_ref, comb_ref, rows_ref,
          sem_g, sem_out):
    wid = lax.axis_index("s") * 2 + lax.axis_index("c")
    iota = lax.iota(jnp.int32, 16)
    tbase = wid * _COMB

    def run_chunk(k, buf, first):
        base = wid * _BW + k * _C
        pltpu.sync_copy(img_ref.at[pl.ds(2 * base, 2 * _C)],
                        pairs_ref.at[buf])
        for i in range(_NGRP):
            e = plsc.load_gather(pairs_ref.at[buf], [32 * i + 2 * iota])
            c = plsc.load_gather(pairs_ref.at[buf], [32 * i + 2 * iota + 1])
            comb_ref[buf, i // 8, pl.ds(16 * (i % 8), 16)] = (
                tbase + e * _NUM_COLORS + c)
        if not first:
            pltpu.make_async_copy(
                rows_ref.at[buf],
                out_ref.at[pl.ds((k - _NBUF) * _C + wid * _BW, _C)],
                sem_out).wait()
        copies = [pltpu.make_async_copy(
            tcomb_ref.at[comb_ref.at[buf, j]],
            rows_ref.at[buf, pl.ds(128 * j, 128)], sem_g)
            for j in range(_NSUB)]
        for cp in copies:
            cp.start()
        for cp in copies:
            cp.wait()
        pltpu.make_async_copy(rows_ref.at[buf],
                              out_ref.at[pl.ds(base, _C)], sem_out).start()

    for b in range(_NBUF):
        run_chunk(b, b, True)

    def outer(g, _):
        for b in range(_NBUF):
            run_chunk(_NBUF * g + b, b, False)
        return ()

    lax.fori_loop(1, _NCHUNK // _NBUF, outer, ())

    for b in range(_NBUF):
        k = _NCHUNK - _NBUF + b
        pltpu.make_async_copy(
            rows_ref.at[b],
            out_ref.at[pl.ds(k * _C + wid * _BW, _C)], sem_out).wait()


@jax.jit
def _encode(img_flat, tcomb):
    mesh = plsc.VectorSubcoreMesh(core_axis_name="c", subcore_axis_name="s")
    return pl.kernel(
        _body,
        out_type=jax.ShapeDtypeStruct((_N, 2 * _EMB), jnp.float32),
        mesh=mesh,
        scratch_types=[
            pltpu.VMEM((_NBUF, 2 * _C), jnp.int32),
            pltpu.VMEM((_NBUF, _NSUB, 128), jnp.int32),
            pltpu.VMEM((_NBUF, _C, 2 * _EMB), jnp.float32),
            pltpu.SemaphoreType.DMA,
            pltpu.SemaphoreType.DMA,
        ],
        compiler_params=pltpu.CompilerParams(
            needs_layout_passes=False, use_tc_tiling_on_sc=False),
    )(img_flat, tcomb)


def kernel(img, entity_table, color_table):
    tcomb = jnp.concatenate(
        [jnp.repeat(entity_table, _NUM_COLORS, axis=0),
         jnp.tile(color_table, (_NUM_TILES, 1))], axis=1)
    tcomb_rep = jnp.tile(tcomb, (_NW, 1))
    out = _encode(img.reshape(-1), tcomb_rep)
    return out.reshape(img.shape[:-1] + (2 * _EMB,))


# speedup vs baseline: 30.7799x; 6.7143x over previous
import jax
import jax.numpy as jnp
from jax import lax
from jax.experimental import pallas as pl
from jax.experimental.pallas import tpu as pltpu
from jax.experimental.pallas import tpu_sc as plsc

_P = 81
_B = 16384
_NW = 32
_TQ = 32
_NU = _P * 4


def _body(img_ref, tbl_ref, out_ref, tblv, inbuf, stage, sem):
    wid = lax.axis_index("s") * 2 + lax.axis_index("c")
    pltpu.sync_copy(tbl_ref, tblv)
    nu = (_NU + _NW - 1 - wid) // _NW

    def unit(ui, _):
        u = wid + _NW * ui
        p = u // 4
        t0 = _TQ * (u - 4 * p)
        pltpu.sync_copy(img_ref.at[p, pl.ds(t0, _TQ)], inbuf)
        for r in range(4):
            ch = 0 if r < 2 else 1

            def tchunk(tt, _, r=r, ch=ch):
                idx = [inbuf[tt, ch, pl.ds(16 * g, 16)] for g in range(8)]
                for s in range(8):
                    base = (r * 8 + s) * 16
                    for g in range(8):
                        stage[tt, s, pl.ds(16 * g, 16)] = plsc.load_gather(
                            tblv, [idx[g] + base])
                return ()

            lax.fori_loop(0, _TQ, tchunk, ())
            pltpu.sync_copy(stage, out_ref.at[p, r, pl.ds(t0, _TQ)])
        return ()

    lax.fori_loop(0, nu, unit, ())


@jax.jit
def _encode(img_lin, tbl):
    mesh = plsc.VectorSubcoreMesh(core_axis_name="c", subcore_axis_name="s")
    return pl.kernel(
        _body,
        out_type=jax.ShapeDtypeStruct((_P, 4, 128, 8, 128), jnp.float32),
        mesh=mesh,
        scratch_types=[
            pltpu.VMEM((512,), jnp.float32),
            pltpu.VMEM((_TQ, 2, 128), jnp.int32),
            pltpu.VMEM((_TQ, 8, 128), jnp.float32),
            pltpu.SemaphoreType.DMA,
        ],
        compiler_params=pltpu.CompilerParams(
            needs_layout_passes=False, use_tc_tiling_on_sc=False),
    )(img_lin, tbl)


def kernel(img, entity_table, color_table):
    tbl = jnp.concatenate(
        [entity_table.T, jnp.pad(color_table.T, ((0, 0), (0, 4)))],
        axis=0).reshape(-1)
    img_lin = jnp.transpose(img, (1, 2, 0, 3)).reshape(9, 9, 128, 128, 2)
    img_lin = jnp.transpose(img_lin, (0, 1, 2, 4, 3)).reshape(_P, 128, 2, 128)
    out_lin = _encode(img_lin, tbl)
    return jnp.transpose(out_lin.reshape(9, 9, 4, 128, 8, 128),
                         (3, 5, 0, 1, 2, 4)).reshape(_B, 9, 9, 32)


# speedup vs baseline: 122.5006x; 3.9799x over previous
import jax
import jax.numpy as jnp
from jax import lax
from jax.experimental import pallas as pl
from jax.experimental.pallas import tpu as pltpu
from jax.experimental.pallas import tpu_sc as plsc

_P = 81
_B = 16384
_NW = 32
_TQ = 32
_NU = 10


def _gather_slab(tblv, inb, st, r):
    ch = 0 if r < 2 else 1

    def tchunk(tt, _):
        idx = [inb[tt, ch, pl.ds(16 * g, 16)] for g in range(8)]
        for s in range(8):
            base = (r * 8 + s) * 16
            for g in range(8):
                st[tt, s, pl.ds(16 * g, 16)] = plsc.load_gather(
                    tblv, [idx[g] + base])
        return ()

    return tchunk


def _body(img_ref, tbl_ref, out_ref, tblv, inbuf, stage, sem_in, sem_out):
    wid = lax.axis_index("s") * 2 + lax.axis_index("c")
    pltpu.sync_copy(tbl_ref, tblv)

    def in_copy(ui, ib):
        u = wid + _NW * ui
        p = u // 4
        t0 = _TQ * (u - 4 * p)
        return (pltpu.make_async_copy(
            img_ref.at[p, pl.ds(t0, _TQ)], inbuf.at[ib], sem_in), p, t0)

    in_copy(0, 0)[0].start()

    def unit(ui, _):
        ib = ui % 2
        cp, p, t0 = in_copy(ui, ib)
        cp.wait()

        @pl.when(ui + 1 < _NU)
        def _prefetch():
            in_copy(ui + 1, 1 - ib)[0].start()

        for r in range(4):
            sb = r % 2
            out_slab = out_ref.at[p, r, pl.ds(t0, _TQ)]
            ring_wait = pltpu.make_async_copy(stage.at[sb], out_slab, sem_out)
            if r < 2:
                @pl.when(ui > 0)
                def _drain():
                    ring_wait.wait()
            else:
                ring_wait.wait()
            lax.fori_loop(0, _TQ,
                          _gather_slab(tblv, inbuf.at[ib], stage.at[sb], r),
                          ())
            pltpu.make_async_copy(stage.at[sb], out_slab, sem_out).start()
        return ()

    lax.fori_loop(0, _NU, unit, ())
    for sb in range(2):
        pltpu.make_async_copy(
            stage.at[sb], out_ref.at[0, 0, pl.ds(0, _TQ)], sem_out).wait()

    t0 = 4 * wid
    pltpu.sync_copy(img_ref.at[_P - 1, pl.ds(t0, 4)],
                    inbuf.at[0, pl.ds(0, 4)])
    for r in range(4):
        lax.fori_loop(0, 4,
                      _gather_slab(tblv, inbuf.at[0], stage.at[0], r), ())
        pltpu.sync_copy(stage.at[0, pl.ds(0, 4)],
                        out_ref.at[_P - 1, r, pl.ds(t0, 4)])


@jax.jit
def _encode(img_lin, tbl):
    mesh = plsc.VectorSubcoreMesh(core_axis_name="c", subcore_axis_name="s")
    return pl.kernel(
        _body,
        out_type=jax.ShapeDtypeStruct((_P, 4, 128, 8, 128), jnp.float32),
        mesh=mesh,
        scratch_types=[
            pltpu.VMEM((512,), jnp.float32),
            pltpu.VMEM((2, _TQ, 2, 128), jnp.int32),
            pltpu.VMEM((2, _TQ, 8, 128), jnp.float32),
            pltpu.SemaphoreType.DMA,
            pltpu.SemaphoreType.DMA,
        ],
        compiler_params=pltpu.CompilerParams(
            needs_layout_passes=False, use_tc_tiling_on_sc=False),
    )(img_lin, tbl)


def kernel(img, entity_table, color_table):
    tbl = jnp.concatenate(
        [entity_table.T, jnp.pad(color_table.T, ((0, 0), (0, 4)))],
        axis=0).reshape(-1)
    img_lin = jnp.transpose(img, (1, 2, 0, 3)).reshape(9, 9, 128, 128, 2)
    img_lin = jnp.transpose(img_lin, (0, 1, 2, 4, 3)).reshape(_P, 128, 2, 128)
    out_lin = _encode(img_lin, tbl)
    return jnp.transpose(out_lin.reshape(9, 9, 4, 128, 8, 128),
                         (3, 5, 0, 1, 2, 4)).reshape(_B, 9, 9, 32)
